# Initial kernel scaffold; baseline (speedup 1.0000x reference)
import jax
import jax.numpy as jnp
from jax.experimental import pallas as pl


def kernel(input, embs, weights, label):
    raise NotImplementedError("write your pallas kernel here")


# speedup vs baseline: 15.9389x; 15.9389x over previous
import functools

import numpy as np
import jax
import jax.numpy as jnp
from jax import lax
from jax.experimental import pallas as pl
from jax.experimental.pallas import tpu as pltpu
from jax.experimental.pallas import tpu_sc as plsc

N_NODES = 1000000
N_NEG = 20
N_SC = N_NEG + 1
D = 64
B = 16384

NC, NS = 2, 16
NW = NC * NS
PER_W = B // NW
C = 32
T = PER_W // C
ROWS = C * N_SC
NEG_C = C * N_NEG
NG = NEG_C // 128
LOG2_N1 = float(np.log2(np.float64(N_NODES + 1)))
LN2 = float(np.log(2.0))


def _sc_scores(embs, weights, label, u):
    mesh = plsc.VectorSubcoreMesh(
        core_axis_name="c", subcore_axis_name="s",
        num_cores=NC, num_subcores=NS)

    @functools.partial(
        pl.kernel,
        out_type=jax.ShapeDtypeStruct((B * N_SC,), jnp.float32),
        mesh=mesh,
        compiler_params=pltpu.CompilerParams(
            needs_layout_passes=False, use_tc_tiling_on_sc=False),
        scratch_types=(
            [pltpu.VMEM((C,), jnp.int32)]
            + [pltpu.VMEM((128,), jnp.int32)] * NG
            + [
                pltpu.VMEM((NEG_C,), jnp.float32),
                pltpu.VMEM((C, D), jnp.float32),
                pltpu.VMEM((ROWS, D), jnp.float32),
                pltpu.VMEM((ROWS,), jnp.float32),
                pltpu.SemaphoreType.DMA,
            ]
        ),
    )
    def k(embs_hbm, w_hbm, lbl_hbm, u_hbm, out_hbm,
          lbl_v, i0, i1, i2, i3, i4, u_v, emb_v, rows_v, sc_v, sem):
        idxs = [i0, i1, i2, i3, i4]
        wid = lax.axis_index("s") * NC + lax.axis_index("c")

        def chunk(t, _):
            base = wid * PER_W + t * C
            pltpu.sync_copy(lbl_hbm.at[pl.ds(base, C)], lbl_v)
            pltpu.sync_copy(u_hbm.at[pl.ds(base * N_NEG, NEG_C)], u_v)
            pltpu.sync_copy(embs_hbm.at[pl.ds(base, C), :], emb_v)
            for kk in range(NEG_C // 16):
                uv = u_v[pl.ds(kk * 16, 16)]
                t = uv * LOG2_N1
                i = (t + 0.5).astype(jnp.int32)
                g = (t - i.astype(jnp.float32)) * LN2
                p = 1.0 + g * (1.0 + g * (0.5 + g * (
                    1.0 / 6 + g * (1.0 / 24 + g * (1.0 / 120 + g / 720)))))
                x = lax.bitcast_convert_type((i + 127) << 23, jnp.float32) * p
                ti = x.astype(jnp.int32)
                up = jnp.where(x > ti.astype(jnp.float32), 1, 0)
                iv = jnp.clip(ti - 2 + up, 0, N_NODES - 1)
                idxs[kk // 8][pl.ds((kk % 8) * 16, 16)] = iv
            cps = [pltpu.async_copy(w_hbm.at[lbl_v],
                                    rows_v.at[pl.ds(0, C)], sem)]
            for g in range(NG):
                cps.append(pltpu.async_copy(
                    w_hbm.at[idxs[g]],
                    rows_v.at[pl.ds(C + g * 128, 128)], sem))
            for cp in cps:
                cp.wait()

            iota = lax.iota(jnp.int32, 16)
            zero = jnp.zeros((16,), jnp.float32)
            for g in range(C // 16):
                srow = g * 16 + iota
                nrow = C + srow * N_NEG

                def dbody(d, accs, srow=srow, nrow=nrow):
                    dv = jnp.zeros((16,), jnp.int32) + d
                    ev = plsc.load_gather(emb_v, [srow, dv])
                    pv = plsc.load_gather(rows_v, [srow, dv])
                    news = [accs[0] + ev * pv]
                    for j in range(N_NEG):
                        nv = plsc.load_gather(rows_v, [nrow + j, dv])
                        news.append(accs[1 + j] + ev * nv)
                    return tuple(news)

                accs = lax.fori_loop(0, D, dbody, (zero,) * N_SC, unroll=4)
                sc_v[pl.ds(g * 16, 16)] = accs[0]
                for j in range(N_NEG):
                    sc_v[pl.ds(C + j * C + g * 16, 16)] = -accs[1 + j]

            pltpu.sync_copy(sc_v, out_hbm.at[pl.ds(base * N_SC, ROWS)])
            return 0

        lax.fori_loop(0, T, chunk, 0)

    return k(embs, weights, label, u)


def _tc_loss(scores):
    def k(x_ref, o_ref):
        x = x_ref[...]
        ls = jnp.minimum(x, 0.0) - jnp.log1p(jnp.exp(-jnp.abs(x)))
        o_ref[0, 0] = -jnp.sum(ls) / B

    return pl.pallas_call(
        k,
        out_shape=jax.ShapeDtypeStruct((1, 1), jnp.float32),
        out_specs=pl.BlockSpec(memory_space=pltpu.SMEM),
    )(scores.reshape(B * N_SC // 128, 128))[0, 0]


def kernel(input, embs, weights, label):
    del input
    u = jax.random.uniform(jax.random.key(42), (N_NEG * B,), dtype=jnp.float32)
    scores = _sc_scores(embs, weights, label, u)
    return _tc_loss(scores)


# speedup vs baseline: 16.9306x; 1.0622x over previous
import functools

import numpy as np
import jax
import jax.numpy as jnp
from jax import lax
from jax.experimental import pallas as pl
from jax.experimental.pallas import tpu as pltpu
from jax.experimental.pallas import tpu_sc as plsc

N_NODES = 1000000
N_NEG = 20
N_SC = N_NEG + 1
D = 64
B = 16384

NC, NS = 2, 16
NW = NC * NS
PER_W = B // NW
C = 32
T = PER_W // C
ROWS = C * N_SC
NEG_C = C * N_NEG
NG = NEG_C // 128
LOG2_N1 = float(np.log2(np.float64(N_NODES + 1)))
LN2 = float(np.log(2.0))


def _sc_scores(embs, weights, label, u):
    mesh = plsc.VectorSubcoreMesh(
        core_axis_name="c", subcore_axis_name="s",
        num_cores=NC, num_subcores=NS)

    @functools.partial(
        pl.kernel,
        out_type=jax.ShapeDtypeStruct((B * N_SC,), jnp.float32),
        mesh=mesh,
        compiler_params=pltpu.CompilerParams(
            needs_layout_passes=False, use_tc_tiling_on_sc=False),
        scratch_types=(
            [pltpu.VMEM((C,), jnp.int32)]
            + [pltpu.VMEM((128,), jnp.int32)] * NG
            + [
                pltpu.VMEM((NEG_C,), jnp.float32),
                pltpu.VMEM((C, D), jnp.float32),
                pltpu.VMEM((ROWS, D), jnp.float32),
                pltpu.VMEM((ROWS,), jnp.float32),
                pltpu.SemaphoreType.DMA,
            ]
        ),
    )
    def k(embs_hbm, w_hbm, lbl_hbm, u_hbm, out_hbm,
          lbl_v, i0, i1, i2, i3, i4, u_v, emb_v, rows_v, sc_v, sem):
        idxs = [i0, i1, i2, i3, i4]
        wid = lax.axis_index("s") * NC + lax.axis_index("c")

        def chunk(t, _):
            base = wid * PER_W + t * C
            pltpu.sync_copy(lbl_hbm.at[pl.ds(base, C)], lbl_v)
            pltpu.sync_copy(u_hbm.at[pl.ds(base * N_NEG, NEG_C)], u_v)
            pltpu.sync_copy(embs_hbm.at[pl.ds(base, C), :], emb_v)
            for kk in range(NEG_C // 16):
                uv = u_v[pl.ds(kk * 16, 16)]
                t = uv * LOG2_N1
                i = (t + 0.5).astype(jnp.int32)
                g = (t - i.astype(jnp.float32)) * LN2
                p = 1.0 + g * (1.0 + g * (0.5 + g * (
                    1.0 / 6 + g * (1.0 / 24 + g * (1.0 / 120 + g / 720)))))
                x = lax.bitcast_convert_type((i + 127) << 23, jnp.float32) * p
                ti = x.astype(jnp.int32)
                up = jnp.where(x > ti.astype(jnp.float32), 1, 0)
                iv = jnp.clip(ti - 2 + up, 0, N_NODES - 1)
                idxs[kk // 8][pl.ds((kk % 8) * 16, 16)] = iv
            cps = [pltpu.async_copy(w_hbm.at[lbl_v],
                                    rows_v.at[pl.ds(0, C)], sem)]
            for g in range(NG):
                cps.append(pltpu.async_copy(
                    w_hbm.at[idxs[g]],
                    rows_v.at[pl.ds(C + g * 128, 128)], sem))
            for cp in cps:
                cp.wait()

            iota = lax.iota(jnp.int32, 16)
            zero = jnp.zeros((16,), jnp.float32)
            for g in range(C // 16):
                srow = g * 16 + iota
                nrow = C + srow * N_NEG

                def dbody(d, accs, srow=srow, nrow=nrow):
                    dv = (iota + d) & (D - 1)
                    ev = plsc.load_gather(emb_v, [srow, dv])
                    pv = plsc.load_gather(rows_v, [srow, dv])
                    news = [accs[0] + ev * pv]
                    for j in range(N_NEG):
                        nv = plsc.load_gather(rows_v, [nrow + j, dv])
                        news.append(accs[1 + j] + ev * nv)
                    return tuple(news)

                accs = lax.fori_loop(0, D, dbody, (zero,) * N_SC, unroll=4)
                sc_v[pl.ds(g * 16, 16)] = accs[0]
                for j in range(N_NEG):
                    sc_v[pl.ds(C + j * C + g * 16, 16)] = -accs[1 + j]

            pltpu.sync_copy(sc_v, out_hbm.at[pl.ds(base * N_SC, ROWS)])
            return 0

        lax.fori_loop(0, T, chunk, 0)

    return k(embs, weights, label, u)


def _tc_loss(scores):
    def k(x_ref, o_ref):
        x = x_ref[...]
        ls = jnp.minimum(x, 0.0) - jnp.log1p(jnp.exp(-jnp.abs(x)))
        o_ref[0, 0] = -jnp.sum(ls) / B

    return pl.pallas_call(
        k,
        out_shape=jax.ShapeDtypeStruct((1, 1), jnp.float32),
        out_specs=pl.BlockSpec(memory_space=pltpu.SMEM),
    )(scores.reshape(B * N_SC // 128, 128))[0, 0]


def kernel(input, embs, weights, label):
    del input
    u = jax.random.uniform(jax.random.key(42), (N_NEG * B,), dtype=jnp.float32)
    scores = _sc_scores(embs, weights, label, u)
    return _tc_loss(scores)
